# Initial kernel scaffold; baseline (speedup 1.0000x reference)
import functools

import jax
import jax.numpy as jnp
from jax import lax
from jax.experimental import pallas as pl
from jax.experimental.pallas import tpu as pltpu
from jax.experimental.pallas import tpu_sc as plsc

B, C, H, W = 8, 3, 512, 512
NCH = B * C
NPIX = H * W
BINS = 256
NC, NS, L = 2, 16, 16
NW = NC * NS
QTR = NPIX // 4
NTASK = NCH * 4
TPW = NTASK // NW
VPQ = QTR // L
_UNROLL = 8

_BC_ROWS = [(b * c) for b in range(B) for c in range(C)]


def _hist_kernel(dst_hbm, ref_hbm, out_hbm, px_v, hist16_v, row_v):
    wid = lax.axis_index("s") * NC + lax.axis_index("c")
    lane = jax.lax.iota(jnp.int32, L)
    laneoff = lane * BINS
    ones = jnp.ones((L,), jnp.float32)
    zeros = jnp.zeros((L,), jnp.float32)

    for img, src in enumerate((dst_hbm, ref_hbm)):
        for k in range(TPW):
            t = wid * TPW + k
            ch = t // 4
            q = lax.rem(t, 4)
            pltpu.sync_copy(src.at[pl.ds(ch * NPIX + q * QTR, QTR)], px_v)

            def _zero(i, _):
                hist16_v[pl.ds(i * L, L)] = zeros
                return 0
            lax.fori_loop(0, (L * BINS) // L, _zero, 0)

            def _acc(i, _):
                for u in range(_UNROLL):
                    x = px_v[pl.ds((i * _UNROLL + u) * L, L)]
                    bn = (x * 256.0).astype(jnp.int32)
                    plsc.addupdate_scatter(hist16_v, [bn + laneoff], ones)
                return 0
            lax.fori_loop(0, VPQ // _UNROLL, _acc, 0)

            def _red(j, _):
                acc = hist16_v[pl.ds(j * L, L)]
                for r in range(1, L):
                    acc = acc + hist16_v[pl.ds(r * BINS + j * L, L)]
                row_v[pl.ds(j * L, L)] = acc
                return 0
            lax.fori_loop(0, BINS // L, _red, 0)

            ch48 = ch + img * NCH
            pltpu.sync_copy(row_v, out_hbm.at[pl.ds((q * 2 * NCH + ch48) * BINS, BINS)])


def _table_kernel(hp_ref, lut_ref, tbl_s):
    hp = hp_ref[...]
    h = jnp.sum(hp, axis=0)
    hn = h * (1.0 / float(NPIX))
    ri = lax.broadcasted_iota(jnp.int32, (BINS, BINS), 0)
    ci = lax.broadcasted_iota(jnp.int32, (BINS, BINS), 1)
    triu = (ri <= ci).astype(jnp.float32)
    cdf = jnp.dot(hn, triu, preferred_element_type=jnp.float32)
    cd = cdf[0:NCH]
    cr = cdf[NCH:2 * NCH]

    def body(i, tbl):
        cri = lax.dynamic_slice(cr, (0, i), (NCH, 1))
        return tbl + (cd >= cri).astype(jnp.float32)
    tbl = lax.fori_loop(0, BINS, body, jnp.zeros((NCH, BINS), jnp.float32))
    tbl_s[...] = jnp.clip(tbl - 1.0, 0.0, 255.0) * (1.0 / 255.0)
    for ch in range(NCH):
        lut_ref[ch, :] = tbl_s[_BC_ROWS[ch], :]


def _apply_kernel(dst_hbm, lut_hbm, out_hbm, px_v, lut_v):
    wid = lax.axis_index("s") * NC + lax.axis_index("c")
    for k in range(TPW):
        t = wid * TPW + k
        ch = t // 4
        q = lax.rem(t, 4)
        pltpu.sync_copy(lut_hbm.at[pl.ds(ch * BINS, BINS)], lut_v)
        off = ch * NPIX + q * QTR
        pltpu.sync_copy(dst_hbm.at[pl.ds(off, QTR)], px_v)

        def _gath(i, _):
            for u in range(_UNROLL):
                sl = pl.ds((i * _UNROLL + u) * L, L)
                idx = (px_v[sl] * 255.0).astype(jnp.int32)
                px_v[sl] = plsc.load_gather(lut_v, [idx])
            return 0
        lax.fori_loop(0, VPQ // _UNROLL, _gath, 0)
        pltpu.sync_copy(px_v, out_hbm.at[pl.ds(off, QTR)])


def kernel(dst, ref):
    dflat = dst.reshape(-1)
    rflat = ref.reshape(-1)
    mesh = plsc.VectorSubcoreMesh(
        core_axis_name="c", subcore_axis_name="s", num_cores=NC, num_subcores=NS)

    hist_parts = pl.kernel(
        _hist_kernel,
        out_type=jax.ShapeDtypeStruct((4 * 2 * NCH * BINS,), jnp.float32),
        mesh=mesh,
        scratch_types=[
            pltpu.VMEM((QTR,), jnp.float32),
            pltpu.VMEM((L * BINS,), jnp.float32),
            pltpu.VMEM((BINS,), jnp.float32),
        ],
    )(dflat, rflat)

    lut = pl.pallas_call(
        _table_kernel,
        out_shape=jax.ShapeDtypeStruct((NCH, BINS), jnp.float32),
        scratch_shapes=[pltpu.VMEM((NCH, BINS), jnp.float32)],
    )(hist_parts.reshape(4, 2 * NCH, BINS))

    out = pl.kernel(
        _apply_kernel,
        out_type=jax.ShapeDtypeStruct((NCH * NPIX,), jnp.float32),
        mesh=mesh,
        scratch_types=[
            pltpu.VMEM((QTR,), jnp.float32),
            pltpu.VMEM((BINS,), jnp.float32),
        ],
    )(dflat, lut.reshape(-1))

    return out.reshape(dst.shape)


# speedup vs baseline: 189.8912x; 189.8912x over previous
import functools

import jax
import jax.numpy as jnp
from jax import lax
from jax.experimental import pallas as pl
from jax.experimental.pallas import tpu as pltpu
from jax.experimental.pallas import tpu_sc as plsc

B, C, H, W = 8, 3, 512, 512
NCH = B * C
NPIX = H * W
BINS = 256
NC, NS, L = 2, 16, 16
NW = NC * NS
QTR = NPIX // 4
NTASK = NCH * 4
TPW = NTASK // NW
VPQ = QTR // L
_UNROLL = 8

_BC_ROWS = [(b * c) for b in range(B) for c in range(C)]


def _hist_kernel(dst_hbm, ref_hbm, out_hbm, px_v, hist16_v, row_v):
    wid = lax.axis_index("s") * NC + lax.axis_index("c")
    lane = jax.lax.iota(jnp.int32, L)
    laneoff = lane * BINS
    ones = jnp.ones((L,), jnp.float32)
    zeros = jnp.zeros((L,), jnp.float32)

    for img, src in enumerate((dst_hbm, ref_hbm)):
        for k in range(TPW):
            t = wid * TPW + k
            ch = t // 4
            q = lax.rem(t, 4)
            pltpu.sync_copy(src.at[pl.ds(ch * NPIX + q * QTR, QTR)], px_v)

            def _zero(i, _):
                hist16_v[pl.ds(i * L, L)] = zeros
                return 0
            lax.fori_loop(0, (L * BINS) // L, _zero, 0)

            def _acc(i, _):
                for u in range(_UNROLL):
                    x = px_v[pl.ds((i * _UNROLL + u) * L, L)]
                    bn = (x * 256.0).astype(jnp.int32)
                    plsc.addupdate_scatter(hist16_v, [bn + laneoff], ones)
                return 0
            lax.fori_loop(0, VPQ // _UNROLL, _acc, 0)

            def _red(j, _):
                acc = hist16_v[pl.ds(j * L, L)]
                for r in range(1, L):
                    acc = acc + hist16_v[pl.ds(r * BINS + j * L, L)]
                row_v[pl.ds(j * L, L)] = acc
                return 0
            lax.fori_loop(0, BINS // L, _red, 0)

            ch48 = ch + img * NCH
            pltpu.sync_copy(row_v, out_hbm.at[pl.ds((q * 2 * NCH + ch48) * BINS, BINS)])


def _table_kernel(hp_ref, lut_ref):
    hp = hp_ref[...]
    h = jnp.sum(hp, axis=0)
    hn = h * (1.0 / float(NPIX))
    ri = lax.broadcasted_iota(jnp.int32, (BINS, BINS), 0)
    ci = lax.broadcasted_iota(jnp.int32, (BINS, BINS), 1)
    triu = (ri <= ci).astype(jnp.float32)
    cdf = jnp.dot(hn, triu, preferred_element_type=jnp.float32)
    cd = cdf[0:NCH]
    ident = (ri == ci).astype(jnp.float32)
    crT = lax.dot_general(ident, cdf[NCH:2 * NCH], (((1,), (1,)), ((), ())),
                          preferred_element_type=jnp.float32)
    for r in sorted(set(_BC_ROWS)):
        col = crT[:, r:r + 1]
        cnt = jnp.sum((cd[r:r + 1, :] >= col).astype(jnp.float32),
                      axis=0, keepdims=True)
        row = jnp.clip(cnt - 1.0, 0.0, 255.0) * (1.0 / 255.0)
        for ch in range(NCH):
            if _BC_ROWS[ch] == r:
                lut_ref[ch:ch + 1, :] = row


def _apply_kernel(dst_hbm, lut_hbm, out_hbm, px_v, lut_v):
    wid = lax.axis_index("s") * NC + lax.axis_index("c")
    for k in range(TPW):
        t = wid * TPW + k
        ch = t // 4
        q = lax.rem(t, 4)
        pltpu.sync_copy(lut_hbm.at[pl.ds(ch * BINS, BINS)], lut_v)
        off = ch * NPIX + q * QTR
        pltpu.sync_copy(dst_hbm.at[pl.ds(off, QTR)], px_v)

        def _gath(i, _):
            for u in range(_UNROLL):
                sl = pl.ds((i * _UNROLL + u) * L, L)
                idx = (px_v[sl] * 255.0).astype(jnp.int32)
                px_v[sl] = plsc.load_gather(lut_v, [idx])
            return 0
        lax.fori_loop(0, VPQ // _UNROLL, _gath, 0)
        pltpu.sync_copy(px_v, out_hbm.at[pl.ds(off, QTR)])


def kernel(dst, ref):
    dflat = dst.reshape(-1)
    rflat = ref.reshape(-1)
    mesh = plsc.VectorSubcoreMesh(
        core_axis_name="c", subcore_axis_name="s", num_cores=NC, num_subcores=NS)
    sc_params = pltpu.CompilerParams(needs_layout_passes=False)

    hist_parts = pl.kernel(
        _hist_kernel,
        out_type=jax.ShapeDtypeStruct((4 * 2 * NCH * BINS,), jnp.float32),
        mesh=mesh,
        compiler_params=sc_params,
        scratch_types=[
            pltpu.VMEM((QTR,), jnp.float32),
            pltpu.VMEM((L * BINS,), jnp.float32),
            pltpu.VMEM((BINS,), jnp.float32),
        ],
    )(dflat, rflat)

    lut = pl.pallas_call(
        _table_kernel,
        out_shape=jax.ShapeDtypeStruct((NCH, BINS), jnp.float32),
        scratch_shapes=[],
    )(hist_parts.reshape(4, 2 * NCH, BINS))

    out = pl.kernel(
        _apply_kernel,
        out_type=jax.ShapeDtypeStruct((NCH * NPIX,), jnp.float32),
        mesh=mesh,
        compiler_params=sc_params,
        scratch_types=[
            pltpu.VMEM((QTR,), jnp.float32),
            pltpu.VMEM((BINS,), jnp.float32),
        ],
    )(dflat, lut.reshape(-1))

    return out.reshape(dst.shape)


# speedup vs baseline: 344.2979x; 1.8131x over previous
import functools

import jax
import jax.numpy as jnp
from jax import lax
from jax.experimental import pallas as pl
from jax.experimental.pallas import tpu as pltpu
from jax.experimental.pallas import tpu_sc as plsc

B, C, H, W = 8, 3, 512, 512
NCH = B * C
NPIX = H * W
BINS = 256
NC, NS, L = 2, 16, 16
NW = NC * NS
QTR = NPIX // 4
NTASK = NCH * 4
TPW = NTASK // NW
VPQ = QTR // L
_UNROLL = 8

_BC_ROWS = [(b * c) for b in range(B) for c in range(C)]


def _hist_kernel(dst_hbm, ref_hbm, out_hbm, px_v, hist16_v, row_v):
    wid = lax.axis_index("s") * NC + lax.axis_index("c")
    lane = jax.lax.iota(jnp.int32, L)
    laneoff = lane * BINS
    ones = jnp.ones((L,), jnp.float32)
    zeros = jnp.zeros((L,), jnp.float32)

    for img, src in enumerate((dst_hbm, ref_hbm)):
        for k in range(TPW):
            t = wid * TPW + k
            ch = t // 4
            q = lax.rem(t, 4)
            pltpu.sync_copy(src.at[pl.ds(ch * NPIX + q * QTR, QTR)], px_v)

            def _zero(i, _):
                hist16_v[pl.ds(i * L, L)] = zeros
                return 0
            lax.fori_loop(0, (L * BINS) // L, _zero, 0)

            def _acc(i, _):
                xs = [px_v[pl.ds((i * _UNROLL + u) * L, L)]
                      for u in range(_UNROLL)]
                addrs = [(x * 256.0).astype(jnp.int32) + laneoff for x in xs]
                for a in addrs:
                    plsc.addupdate_scatter(hist16_v, [a], ones)
                return 0
            lax.fori_loop(0, VPQ // _UNROLL, _acc, 0)

            def _red(j, _):
                acc = hist16_v[pl.ds(j * L, L)]
                for r in range(1, L):
                    acc = acc + hist16_v[pl.ds(r * BINS + j * L, L)]
                row_v[pl.ds(j * L, L)] = acc
                return 0
            lax.fori_loop(0, BINS // L, _red, 0)

            ch48 = ch + img * NCH
            pltpu.sync_copy(row_v, out_hbm.at[pl.ds((q * 2 * NCH + ch48) * BINS, BINS)])


def _table_kernel(hp_ref, lut_ref):
    hp = hp_ref[...]
    h = jnp.sum(hp, axis=0)
    hn = h * (1.0 / float(NPIX))
    ri = lax.broadcasted_iota(jnp.int32, (BINS, BINS), 0)
    ci = lax.broadcasted_iota(jnp.int32, (BINS, BINS), 1)
    triu = (ri <= ci).astype(jnp.float32)
    cdf = jnp.dot(hn, triu, preferred_element_type=jnp.float32)
    cd = cdf[0:NCH]
    ident = (ri == ci).astype(jnp.float32)
    crT = lax.dot_general(ident, cdf[NCH:2 * NCH], (((1,), (1,)), ((), ())),
                          preferred_element_type=jnp.float32)
    for r in sorted(set(_BC_ROWS)):
        col = crT[:, r:r + 1]
        cnt = jnp.sum((cd[r:r + 1, :] >= col).astype(jnp.float32),
                      axis=0, keepdims=True)
        row = jnp.clip(cnt - 1.0, 0.0, 255.0) * (1.0 / 255.0)
        for ch in range(NCH):
            if _BC_ROWS[ch] == r:
                lut_ref[ch:ch + 1, :] = row


def _apply_kernel(dst_hbm, lut_hbm, out_hbm, px_v, lut_v):
    wid = lax.axis_index("s") * NC + lax.axis_index("c")
    for k in range(TPW):
        t = wid * TPW + k
        ch = t // 4
        q = lax.rem(t, 4)
        pltpu.sync_copy(lut_hbm.at[pl.ds(ch * BINS, BINS)], lut_v)
        off = ch * NPIX + q * QTR
        pltpu.sync_copy(dst_hbm.at[pl.ds(off, QTR)], px_v)

        def _gath(i, _):
            sls = [pl.ds((i * _UNROLL + u) * L, L) for u in range(_UNROLL)]
            idxs = [(px_v[sl] * 255.0).astype(jnp.int32) for sl in sls]
            vals = [plsc.load_gather(lut_v, [idx]) for idx in idxs]
            for sl, v in zip(sls, vals):
                px_v[sl] = v
            return 0
        lax.fori_loop(0, VPQ // _UNROLL, _gath, 0)
        pltpu.sync_copy(px_v, out_hbm.at[pl.ds(off, QTR)])


def kernel(dst, ref):
    dflat = dst.reshape(-1)
    rflat = ref.reshape(-1)
    mesh = plsc.VectorSubcoreMesh(
        core_axis_name="c", subcore_axis_name="s", num_cores=NC, num_subcores=NS)
    sc_params = pltpu.CompilerParams(needs_layout_passes=False)

    hist_parts = pl.kernel(
        _hist_kernel,
        out_type=jax.ShapeDtypeStruct((4 * 2 * NCH * BINS,), jnp.float32),
        mesh=mesh,
        compiler_params=sc_params,
        scratch_types=[
            pltpu.VMEM((QTR,), jnp.float32),
            pltpu.VMEM((L * BINS,), jnp.float32),
            pltpu.VMEM((BINS,), jnp.float32),
        ],
    )(dflat, rflat)

    lut = pl.pallas_call(
        _table_kernel,
        out_shape=jax.ShapeDtypeStruct((NCH, BINS), jnp.float32),
        scratch_shapes=[],
    )(hist_parts.reshape(4, 2 * NCH, BINS))

    out = pl.kernel(
        _apply_kernel,
        out_type=jax.ShapeDtypeStruct((NCH * NPIX,), jnp.float32),
        mesh=mesh,
        compiler_params=sc_params,
        scratch_types=[
            pltpu.VMEM((QTR,), jnp.float32),
            pltpu.VMEM((BINS,), jnp.float32),
        ],
    )(dflat, lut.reshape(-1))

    return out.reshape(dst.shape)


# speedup vs baseline: 412.7827x; 1.1989x over previous
import jax
import jax.numpy as jnp
from jax import lax
from jax.experimental import pallas as pl
from jax.experimental.pallas import tpu as pltpu
from jax.experimental.pallas import tpu_sc as plsc

B, C, H, W = 8, 3, 512, 512
NCH = B * C
NPIX = H * W
BINS = 256
NC, NS, L = 2, 16, 16
NW = NC * NS
QTR = NPIX // 4
NTASK = NCH * 4
TPW = NTASK // NW
CH = 16384
CPQ = QTR // CH
CHV = CH // L
_UNROLL = 8

_BC_ROWS = [(b * c) for b in range(B) for c in range(C)]


def _hist_kernel(dst_hbm, ref_hbm, out_hbm,
                 pxa_v, pxb_v, hist16_v, row_v, sem_a, sem_b):
    wid = lax.axis_index("s") * NC + lax.axis_index("c")
    lane = jax.lax.iota(jnp.int32, L)
    laneoff = lane * BINS
    ones = jnp.ones((L,), jnp.float32)
    zeros = jnp.zeros((L,), jnp.float32)
    bufs = (pxa_v, pxb_v)
    sems = (sem_a, sem_b)

    chunks = [(src, k, j)
              for src in (dst_hbm, ref_hbm)
              for k in range(TPW)
              for j in range(CPQ)]

    def chunk_off(k, j):
        t = wid * TPW + k
        ch = t // CPQ
        q = lax.rem(t, CPQ)
        return ch * NPIX + q * QTR + j * CH

    def start(m):
        src, k, j = chunks[m]
        return pltpu.async_copy(
            src.at[pl.ds(chunk_off(k, j), CH)], bufs[m % 2], sems[m % 2])

    def _zero(i, _):
        hist16_v[pl.ds(i * L, L)] = zeros
        return 0
    lax.fori_loop(0, (L * BINS) // L, _zero, 0)

    pending = start(0)
    for m, (src, k, j) in enumerate(chunks):
        nxt = start(m + 1) if m + 1 < len(chunks) else None
        pending.wait()
        pending = nxt
        buf = bufs[m % 2]

        def _acc(i, _):
            xs = [buf[pl.ds((i * _UNROLL + u) * L, L)]
                  for u in range(_UNROLL)]
            addrs = [(x * 256.0).astype(jnp.int32) + laneoff for x in xs]
            for a in addrs:
                plsc.addupdate_scatter(hist16_v, [a], ones)
            return 0
        lax.fori_loop(0, CHV // _UNROLL, _acc, 0)

        if j == CPQ - 1:
            def _red(jj, _):
                acc = hist16_v[pl.ds(jj * L, L)]
                hist16_v[pl.ds(jj * L, L)] = zeros
                for r in range(1, L):
                    sl = pl.ds(r * BINS + jj * L, L)
                    acc = acc + hist16_v[sl]
                    hist16_v[sl] = zeros
                row_v[pl.ds(jj * L, L)] = acc
                return 0
            lax.fori_loop(0, BINS // L, _red, 0)

            t = wid * TPW + k
            ch = t // CPQ
            q = lax.rem(t, CPQ)
            img = 0 if src is dst_hbm else 1
            ch48 = ch + img * NCH
            pltpu.sync_copy(
                row_v, out_hbm.at[pl.ds((q * 2 * NCH + ch48) * BINS, BINS)])


def _table_kernel(hp_ref, lut_ref):
    hp = hp_ref[...]
    h = jnp.sum(hp, axis=0)
    hn = h * (1.0 / float(NPIX))
    ri = lax.broadcasted_iota(jnp.int32, (BINS, BINS), 0)
    ci = lax.broadcasted_iota(jnp.int32, (BINS, BINS), 1)
    triu = (ri <= ci).astype(jnp.float32)
    cdf = jnp.dot(hn, triu, preferred_element_type=jnp.float32)
    cd = cdf[0:NCH]
    ident = (ri == ci).astype(jnp.float32)
    crT = lax.dot_general(ident, cdf[NCH:2 * NCH], (((1,), (1,)), ((), ())),
                          preferred_element_type=jnp.float32)
    for r in sorted(set(_BC_ROWS)):
        col = crT[:, r:r + 1]
        cnt = jnp.sum((cd[r:r + 1, :] >= col).astype(jnp.float32),
                      axis=0, keepdims=True)
        row = jnp.clip(cnt - 1.0, 0.0, 255.0) * (1.0 / 255.0)
        for ch in range(NCH):
            if _BC_ROWS[ch] == r:
                lut_ref[ch:ch + 1, :] = row


def _apply_kernel(dst_hbm, lut_hbm, out_hbm,
                  px0_v, px1_v, px2_v, px3_v, lut_v, *sems):
    wid = lax.axis_index("s") * NC + lax.axis_index("c")
    bufs = (px0_v, px1_v, px2_v, px3_v)
    in_sems = sems[0:4]
    out_sems = sems[4:8]
    M = TPW * CPQ

    def chunk_off(m):
        k, j = m // CPQ, m % CPQ
        t = wid * TPW + k
        ch = t // CPQ
        q = lax.rem(t, CPQ)
        return ch * NPIX + q * QTR + j * CH

    def start_in(m):
        return pltpu.async_copy(
            dst_hbm.at[pl.ds(chunk_off(m), CH)], bufs[m % 4], in_sems[m % 4])

    def start_out(m):
        return pltpu.async_copy(
            bufs[m % 4], out_hbm.at[pl.ds(chunk_off(m), CH)], out_sems[m % 4])

    in_h = {0: start_in(0), 1: start_in(1)}
    out_h = {}
    for m in range(M):
        if m >= 2:
            out_h.pop(m - 2).wait()
        if m + 2 < M:
            in_h[m + 2] = start_in(m + 2)
        if m % CPQ == 0:
            t = wid * TPW + m // CPQ
            chn = t // CPQ
            pltpu.sync_copy(lut_hbm.at[pl.ds(chn * BINS, BINS)], lut_v)
        in_h.pop(m).wait()
        buf = bufs[m % 4]

        def _gath(i, _):
            sls = [pl.ds((i * _UNROLL + u) * L, L) for u in range(_UNROLL)]
            idxs = [(buf[sl] * 255.0).astype(jnp.int32) for sl in sls]
            vals = [plsc.load_gather(lut_v, [idx]) for idx in idxs]
            for sl, v in zip(sls, vals):
                buf[sl] = v
            return 0
        lax.fori_loop(0, CHV // _UNROLL, _gath, 0)
        out_h[m] = start_out(m)
    for m in sorted(out_h):
        out_h.pop(m).wait()


def kernel(dst, ref):
    dflat = dst.reshape(-1)
    rflat = ref.reshape(-1)
    mesh = plsc.VectorSubcoreMesh(
        core_axis_name="c", subcore_axis_name="s", num_cores=NC, num_subcores=NS)
    sc_params = pltpu.CompilerParams(needs_layout_passes=False)

    hist_parts = pl.kernel(
        _hist_kernel,
        out_type=jax.ShapeDtypeStruct((4 * 2 * NCH * BINS,), jnp.float32),
        mesh=mesh,
        compiler_params=sc_params,
        scratch_types=[
            pltpu.VMEM((CH,), jnp.float32),
            pltpu.VMEM((CH,), jnp.float32),
            pltpu.VMEM((L * BINS,), jnp.float32),
            pltpu.VMEM((BINS,), jnp.float32),
            pltpu.SemaphoreType.DMA,
            pltpu.SemaphoreType.DMA,
        ],
    )(dflat, rflat)

    lut = pl.pallas_call(
        _table_kernel,
        out_shape=jax.ShapeDtypeStruct((NCH, BINS), jnp.float32),
    )(hist_parts.reshape(4, 2 * NCH, BINS))

    out = pl.kernel(
        _apply_kernel,
        out_type=jax.ShapeDtypeStruct((NCH * NPIX,), jnp.float32),
        mesh=mesh,
        compiler_params=sc_params,
        scratch_types=[
            pltpu.VMEM((CH,), jnp.float32),
            pltpu.VMEM((CH,), jnp.float32),
            pltpu.VMEM((CH,), jnp.float32),
            pltpu.VMEM((CH,), jnp.float32),
            pltpu.VMEM((BINS,), jnp.float32),
            pltpu.SemaphoreType.DMA,
            pltpu.SemaphoreType.DMA,
            pltpu.SemaphoreType.DMA,
            pltpu.SemaphoreType.DMA,
            pltpu.SemaphoreType.DMA,
            pltpu.SemaphoreType.DMA,
            pltpu.SemaphoreType.DMA,
            pltpu.SemaphoreType.DMA,
        ],
    )(dflat, lut.reshape(-1))

    return out.reshape(dst.shape)


# speedup vs baseline: 715.2270x; 1.7327x over previous
import jax
import jax.numpy as jnp
from jax import lax
from jax.experimental import pallas as pl
from jax.experimental.pallas import tpu as pltpu
from jax.experimental.pallas import tpu_sc as plsc

B, C, H, W = 8, 3, 512, 512
NCH = B * C
NPIX = H * W
BINS = 256
NC, NS, L = 2, 16, 16
NW = NC * NS
QROWS = H // 4
NTASK = NCH * 4
TPW = NTASK // NW
CROWS = 32
CPQ = QROWS // CROWS
LGR = W // L
_BC_ROWS = [(b * c) for b in range(B) for c in range(C)]


def _px_loop(buf, body16):
    def _row(r, _):
        body16(r)
        return 0
    lax.fori_loop(0, CROWS, _row, 0)


def _hist_kernel(dst_hbm, ref_hbm, out_hbm,
                 pxa_v, pxb_v, hist16_v, row_v, sem_a, sem_b):
    wid = lax.axis_index("s") * NC + lax.axis_index("c")
    lane = jax.lax.iota(jnp.int32, L)
    laneoff = lane * BINS
    ones = jnp.ones((L,), jnp.float32)
    zeros = jnp.zeros((L,), jnp.float32)
    bufs = (pxa_v, pxb_v)
    sems = (sem_a, sem_b)

    chunks = [(src, k, j)
              for src in (dst_hbm, ref_hbm)
              for k in range(TPW)
              for j in range(CPQ)]

    def task_ch_q(k):
        t = wid * TPW + k
        return t // 4, lax.rem(t, 4)

    def start(m):
        src, k, j = chunks[m]
        ch, q = task_ch_q(k)
        r0 = q * QROWS + j * CROWS
        return pltpu.async_copy(
            src.at[ch, pl.ds(r0, CROWS)], bufs[m % 2], sems[m % 2])

    def _zero(i, _):
        hist16_v[pl.ds(i * L, L)] = zeros
        return 0
    lax.fori_loop(0, (L * BINS) // L, _zero, 0)

    pending = start(0)
    for m, (src, k, j) in enumerate(chunks):
        nxt = start(m + 1) if m + 1 < len(chunks) else None
        pending.wait()
        pending = nxt
        buf = bufs[m % 2]

        def _row(r):
            xs = [buf[r, pl.ds(g * L, L)] for g in range(LGR)]
            addrs = [(x * 256.0).astype(jnp.int32) + laneoff for x in xs]
            for a in addrs:
                plsc.addupdate_scatter(hist16_v, [a], ones)
        _px_loop(buf, _row)

        if j == CPQ - 1:
            def _red(jj, _):
                acc = hist16_v[pl.ds(jj * L, L)]
                hist16_v[pl.ds(jj * L, L)] = zeros
                for r in range(1, L):
                    sl = pl.ds(r * BINS + jj * L, L)
                    acc = acc + hist16_v[sl]
                    hist16_v[sl] = zeros
                row_v[pl.ds(jj * L, L)] = acc
                return 0
            lax.fori_loop(0, BINS // L, _red, 0)

            ch, q = task_ch_q(k)
            img = 0 if src is dst_hbm else 1
            ch48 = ch + img * NCH
            pltpu.sync_copy(
                row_v, out_hbm.at[pl.ds((q * 2 * NCH + ch48) * BINS, BINS)])


def _table_kernel(hp_ref, lut_ref):
    hp = hp_ref[...]
    h = jnp.sum(hp, axis=0)
    hn = h * (1.0 / float(NPIX))
    ri = lax.broadcasted_iota(jnp.int32, (BINS, BINS), 0)
    ci = lax.broadcasted_iota(jnp.int32, (BINS, BINS), 1)
    triu = (ri <= ci).astype(jnp.float32)
    cdf = jnp.dot(hn, triu, preferred_element_type=jnp.float32)
    cd = cdf[0:NCH]
    ident = (ri == ci).astype(jnp.float32)
    crT = lax.dot_general(ident, cdf[NCH:2 * NCH], (((1,), (1,)), ((), ())),
                          preferred_element_type=jnp.float32)
    for r in sorted(set(_BC_ROWS)):
        col = crT[:, r:r + 1]
        cnt = jnp.sum((cd[r:r + 1, :] >= col).astype(jnp.float32),
                      axis=0, keepdims=True)
        row = jnp.clip(cnt - 1.0, 0.0, 255.0) * (1.0 / 255.0)
        for ch in range(NCH):
            if _BC_ROWS[ch] == r:
                lut_ref[ch:ch + 1, :] = row


def _apply_kernel(dst_hbm, lut_hbm, out_hbm,
                  px0_v, px1_v, px2_v, px3_v, lut_v, *sems):
    wid = lax.axis_index("s") * NC + lax.axis_index("c")
    bufs = (px0_v, px1_v, px2_v, px3_v)
    in_sems = sems[0:4]
    out_sems = sems[4:8]
    M = TPW * CPQ

    def chunk_pos(m):
        k, j = m // CPQ, m % CPQ
        t = wid * TPW + k
        ch = t // 4
        q = lax.rem(t, 4)
        return ch, q * QROWS + j * CROWS

    def start_in(m):
        ch, r0 = chunk_pos(m)
        return pltpu.async_copy(
            dst_hbm.at[ch, pl.ds(r0, CROWS)], bufs[m % 4], in_sems[m % 4])

    def start_out(m):
        ch, r0 = chunk_pos(m)
        return pltpu.async_copy(
            bufs[m % 4], out_hbm.at[ch, pl.ds(r0, CROWS)], out_sems[m % 4])

    in_h = {0: start_in(0), 1: start_in(1)}
    out_h = {}
    for m in range(M):
        if m >= 2:
            out_h.pop(m - 2).wait()
        if m + 2 < M:
            in_h[m + 2] = start_in(m + 2)
        if m % CPQ == 0:
            t = wid * TPW + m // CPQ
            chn = t // 4
            pltpu.sync_copy(lut_hbm.at[pl.ds(chn * BINS, BINS)], lut_v)
        in_h.pop(m).wait()
        buf = bufs[m % 4]

        def _row(r):
            sls = [pl.ds(g * L, L) for g in range(LGR)]
            idxs = [(buf[r, sl] * 255.0).astype(jnp.int32) for sl in sls]
            vals = [plsc.load_gather(lut_v, [idx]) for idx in idxs]
            for sl, v in zip(sls, vals):
                buf[r, sl] = v
        _px_loop(buf, _row)
        out_h[m] = start_out(m)
    for m in sorted(out_h):
        out_h.pop(m).wait()


def kernel(dst, ref):
    d3 = dst.reshape(NCH, H, W)
    r3 = ref.reshape(NCH, H, W)
    mesh = plsc.VectorSubcoreMesh(
        core_axis_name="c", subcore_axis_name="s", num_cores=NC, num_subcores=NS)
    sc_params = pltpu.CompilerParams(
        needs_layout_passes=False, use_tc_tiling_on_sc=True)

    hist_parts = pl.kernel(
        _hist_kernel,
        out_type=jax.ShapeDtypeStruct((4 * 2 * NCH * BINS,), jnp.float32),
        mesh=mesh,
        compiler_params=sc_params,
        scratch_types=[
            pltpu.VMEM((CROWS, W), jnp.float32),
            pltpu.VMEM((CROWS, W), jnp.float32),
            pltpu.VMEM((L * BINS,), jnp.float32),
            pltpu.VMEM((BINS,), jnp.float32),
            pltpu.SemaphoreType.DMA,
            pltpu.SemaphoreType.DMA,
        ],
    )(d3, r3)

    lut = pl.pallas_call(
        _table_kernel,
        out_shape=jax.ShapeDtypeStruct((NCH, BINS), jnp.float32),
    )(hist_parts.reshape(4, 2 * NCH, BINS))

    out = pl.kernel(
        _apply_kernel,
        out_type=jax.ShapeDtypeStruct((NCH, H, W), jnp.float32),
        mesh=mesh,
        compiler_params=sc_params,
        scratch_types=[
            pltpu.VMEM((CROWS, W), jnp.float32),
            pltpu.VMEM((CROWS, W), jnp.float32),
            pltpu.VMEM((CROWS, W), jnp.float32),
            pltpu.VMEM((CROWS, W), jnp.float32),
            pltpu.VMEM((BINS,), jnp.float32),
            pltpu.SemaphoreType.DMA,
            pltpu.SemaphoreType.DMA,
            pltpu.SemaphoreType.DMA,
            pltpu.SemaphoreType.DMA,
            pltpu.SemaphoreType.DMA,
            pltpu.SemaphoreType.DMA,
            pltpu.SemaphoreType.DMA,
            pltpu.SemaphoreType.DMA,
        ],
    )(d3, lut.reshape(-1))

    return out.reshape(dst.shape)


# speedup vs baseline: 760.0339x; 1.0626x over previous
import jax
import jax.numpy as jnp
from jax import lax
from jax.experimental import pallas as pl
from jax.experimental.pallas import tpu as pltpu
from jax.experimental.pallas import tpu_sc as plsc

B, C, H, W = 8, 3, 512, 512
NCH = B * C
NPIX = H * W
BINS = 256
NC, NS, L = 2, 16, 16
NW = NC * NS
QROWS = H // 4
NTASK = NCH * 4
TPW = NTASK // NW
CROWS = 32
CPQ = QROWS // CROWS
LGR = W // L
_BC_ROWS = [(b * c) for b in range(B) for c in range(C)]


def _px_loop(buf, body16):
    @plsc.parallel_loop(0, CROWS, 1)
    def _row(r):
        body16(r)


def _hist_kernel(dst_hbm, ref_hbm, out_hbm,
                 pxa_v, pxb_v, hist16_v, row_v, sem_a, sem_b):
    wid = lax.axis_index("s") * NC + lax.axis_index("c")
    lane = jax.lax.iota(jnp.int32, L)
    laneoff = lane * BINS
    ones = jnp.ones((L,), jnp.float32)
    zeros = jnp.zeros((L,), jnp.float32)
    bufs = (pxa_v, pxb_v)
    sems = (sem_a, sem_b)

    chunks = [(src, k, j)
              for src in (dst_hbm, ref_hbm)
              for k in range(TPW)
              for j in range(CPQ)]

    def task_ch_q(k):
        t = wid * TPW + k
        return t // 4, lax.rem(t, 4)

    def start(m):
        src, k, j = chunks[m]
        ch, q = task_ch_q(k)
        r0 = q * QROWS + j * CROWS
        return pltpu.async_copy(
            src.at[ch, pl.ds(r0, CROWS)], bufs[m % 2], sems[m % 2])

    def _zero(i, _):
        hist16_v[pl.ds(i * L, L)] = zeros
        return 0
    lax.fori_loop(0, (L * BINS) // L, _zero, 0)

    pending = start(0)
    for m, (src, k, j) in enumerate(chunks):
        nxt = start(m + 1) if m + 1 < len(chunks) else None
        pending.wait()
        pending = nxt
        buf = bufs[m % 2]

        def _row(r):
            xs = [buf[r, pl.ds(g * L, L)] for g in range(LGR)]
            addrs = [(x * 256.0).astype(jnp.int32) + laneoff for x in xs]
            for a in addrs:
                plsc.addupdate_scatter(hist16_v, [a], ones)
        _px_loop(buf, _row)

        if j == CPQ - 1:
            def _red(jj, _):
                acc = hist16_v[pl.ds(jj * L, L)]
                hist16_v[pl.ds(jj * L, L)] = zeros
                for r in range(1, L):
                    sl = pl.ds(r * BINS + jj * L, L)
                    acc = acc + hist16_v[sl]
                    hist16_v[sl] = zeros
                row_v[pl.ds(jj * L, L)] = acc
                return 0
            lax.fori_loop(0, BINS // L, _red, 0)

            ch, q = task_ch_q(k)
            img = 0 if src is dst_hbm else 1
            ch48 = ch + img * NCH
            pltpu.sync_copy(row_v, out_hbm.at[q, ch48])


def _table_kernel(hp_ref, lut_ref):
    hp = hp_ref[...]
    h = jnp.sum(hp, axis=0)
    hn = h * (1.0 / float(NPIX))
    ri = lax.broadcasted_iota(jnp.int32, (BINS, BINS), 0)
    ci = lax.broadcasted_iota(jnp.int32, (BINS, BINS), 1)
    triu = (ri <= ci).astype(jnp.float32)
    cdf = jnp.dot(hn, triu, preferred_element_type=jnp.float32)
    cd = cdf[0:NCH]
    ident = (ri == ci).astype(jnp.float32)
    crT = lax.dot_general(ident, cdf[NCH:2 * NCH], (((1,), (1,)), ((), ())),
                          preferred_element_type=jnp.float32)
    for r in sorted(set(_BC_ROWS)):
        col = crT[:, r:r + 1]
        cnt = jnp.sum((cd[r:r + 1, :] >= col).astype(jnp.float32),
                      axis=0, keepdims=True)
        row = jnp.clip(cnt - 1.0, 0.0, 255.0) * (1.0 / 255.0)
        for ch in range(NCH):
            if _BC_ROWS[ch] == r:
                lut_ref[ch:ch + 1, :] = row


def _apply_kernel(dst_hbm, lut_hbm, out_hbm,
                  px0_v, px1_v, px2_v, px3_v, lut_v, *sems):
    wid = lax.axis_index("s") * NC + lax.axis_index("c")
    bufs = (px0_v, px1_v, px2_v, px3_v)
    in_sems = sems[0:4]
    out_sems = sems[4:8]
    M = TPW * CPQ

    def chunk_pos(m):
        k, j = m // CPQ, m % CPQ
        t = wid * TPW + k
        ch = t // 4
        q = lax.rem(t, 4)
        return ch, q * QROWS + j * CROWS

    def start_in(m):
        ch, r0 = chunk_pos(m)
        return pltpu.async_copy(
            dst_hbm.at[ch, pl.ds(r0, CROWS)], bufs[m % 4], in_sems[m % 4])

    def start_out(m):
        ch, r0 = chunk_pos(m)
        return pltpu.async_copy(
            bufs[m % 4], out_hbm.at[ch, pl.ds(r0, CROWS)], out_sems[m % 4])

    in_h = {0: start_in(0), 1: start_in(1)}
    out_h = {}
    for m in range(M):
        if m >= 2:
            out_h.pop(m - 2).wait()
        if m + 2 < M:
            in_h[m + 2] = start_in(m + 2)
        if m % CPQ == 0:
            t = wid * TPW + m // CPQ
            chn = t // 4
            pltpu.sync_copy(lut_hbm.at[chn], lut_v)
        in_h.pop(m).wait()
        buf = bufs[m % 4]

        def _row(r):
            sls = [pl.ds(g * L, L) for g in range(LGR)]
            idxs = [(buf[r, sl] * 255.0).astype(jnp.int32) for sl in sls]
            vals = [plsc.load_gather(lut_v, [idx]) for idx in idxs]
            for sl, v in zip(sls, vals):
                buf[r, sl] = v
        _px_loop(buf, _row)
        out_h[m] = start_out(m)
    for m in sorted(out_h):
        out_h.pop(m).wait()


def kernel(dst, ref):
    d3 = dst.reshape(NCH, H, W)
    r3 = ref.reshape(NCH, H, W)
    mesh = plsc.VectorSubcoreMesh(
        core_axis_name="c", subcore_axis_name="s", num_cores=NC, num_subcores=NS)
    sc_params = pltpu.CompilerParams(
        needs_layout_passes=False, use_tc_tiling_on_sc=True)

    hist_parts = pl.kernel(
        _hist_kernel,
        out_type=jax.ShapeDtypeStruct((4, 2 * NCH, BINS), jnp.float32),
        mesh=mesh,
        compiler_params=sc_params,
        scratch_types=[
            pltpu.VMEM((CROWS, W), jnp.float32),
            pltpu.VMEM((CROWS, W), jnp.float32),
            pltpu.VMEM((L * BINS,), jnp.float32),
            pltpu.VMEM((BINS,), jnp.float32),
            pltpu.SemaphoreType.DMA,
            pltpu.SemaphoreType.DMA,
        ],
    )(d3, r3)

    lut = pl.pallas_call(
        _table_kernel,
        out_shape=jax.ShapeDtypeStruct((NCH, BINS), jnp.float32),
    )(hist_parts)

    out = pl.kernel(
        _apply_kernel,
        out_type=jax.ShapeDtypeStruct((NCH, H, W), jnp.float32),
        mesh=mesh,
        compiler_params=sc_params,
        scratch_types=[
            pltpu.VMEM((CROWS, W), jnp.float32),
            pltpu.VMEM((CROWS, W), jnp.float32),
            pltpu.VMEM((CROWS, W), jnp.float32),
            pltpu.VMEM((CROWS, W), jnp.float32),
            pltpu.VMEM((BINS,), jnp.float32),
            pltpu.SemaphoreType.DMA,
            pltpu.SemaphoreType.DMA,
            pltpu.SemaphoreType.DMA,
            pltpu.SemaphoreType.DMA,
            pltpu.SemaphoreType.DMA,
            pltpu.SemaphoreType.DMA,
            pltpu.SemaphoreType.DMA,
            pltpu.SemaphoreType.DMA,
        ],
    )(d3, lut)

    return out.reshape(dst.shape)
